# baseline (speedup 1.0000x reference)
import functools

import jax
import jax.numpy as jnp
from jax.experimental import pallas as pl
from jax.experimental.pallas import tpu as pltpu

TOTAL_TOK = 16384
D_IN = 256
D_HIDDEN = 512
D_LATENT = 256
N_BATCHES = 16

BLK = 1024
GRID = TOTAL_TOK // BLK


def _body(x_ref, ids_ref, w1_ref, b1_ref, w2_ref, b2_ref, o_ref, acc_ref, cnt_ref):
    i = pl.program_id(0)

    @pl.when(i == 0)
    def _init():
        acc_ref[...] = jnp.zeros_like(acc_ref)
        cnt_ref[...] = jnp.zeros_like(cnt_ref)

    x = x_ref[...]
    h = jnp.dot(x, w1_ref[...], preferred_element_type=jnp.float32) + b1_ref[...]
    h = jnp.maximum(h, 0.0).astype(jnp.bfloat16)
    y = jnp.dot(h, w2_ref[...], preferred_element_type=jnp.float32) + b2_ref[...]

    ids = ids_ref[0, 0, :]
    seg = jax.lax.broadcasted_iota(jnp.int32, (N_BATCHES, BLK), 0)
    onehot = (seg == ids[None, :]).astype(jnp.float32)
    acc_ref[...] += jnp.dot(onehot, y, preferred_element_type=jnp.float32)
    cnt_ref[...] += jnp.broadcast_to(
        jnp.sum(onehot, axis=1, keepdims=True), cnt_ref.shape
    )

    @pl.when(i == GRID - 1)
    def _fin():
        o_ref[...] = acc_ref[...] / jnp.maximum(cnt_ref[...], 1.0)


@jax.jit
def kernel(x_flat, batch, W1, b1, W2, b2):
    ids3 = batch.reshape(GRID, 1, BLK)
    b1r = b1.reshape(1, D_HIDDEN)
    b2r = b2.reshape(1, D_LATENT)
    xb = x_flat.astype(jnp.bfloat16)
    W1b = W1.astype(jnp.bfloat16)
    W2b = W2.astype(jnp.bfloat16)
    return pl.pallas_call(
        _body,
        grid=(GRID,),
        in_specs=[
            pl.BlockSpec((BLK, D_IN), lambda i: (i, 0)),
            pl.BlockSpec((1, 1, BLK), lambda i: (i, 0, 0)),
            pl.BlockSpec((D_IN, D_HIDDEN), lambda i: (0, 0)),
            pl.BlockSpec((1, D_HIDDEN), lambda i: (0, 0)),
            pl.BlockSpec((D_HIDDEN, D_LATENT), lambda i: (0, 0)),
            pl.BlockSpec((1, D_LATENT), lambda i: (0, 0)),
        ],
        out_specs=pl.BlockSpec((N_BATCHES, D_LATENT), lambda i: (0, 0)),
        out_shape=jax.ShapeDtypeStruct((N_BATCHES, D_LATENT), jnp.float32),
        scratch_shapes=[
            pltpu.VMEM((N_BATCHES, D_LATENT), jnp.float32),
            pltpu.VMEM((N_BATCHES, D_LATENT), jnp.float32),
        ],
    )(xb, ids3, W1b, b1r, W2b, b2r)


# speedup vs baseline: 1.8383x; 1.8383x over previous
import functools

import jax
import jax.numpy as jnp
from jax.experimental import pallas as pl
from jax.experimental.pallas import tpu as pltpu

TOTAL_TOK = 16384
D_IN = 256
D_HIDDEN = 512
D_LATENT = 256
N_BATCHES = 16

BLK = 2048
HALF = BLK // 2
GRID = TOTAL_TOK // BLK


def _body(x_ref, ids_ref, w1_ref, b1_ref, w2_ref, b2_ref, o_ref, acc_ref, cnt_ref):
    i = pl.program_id(0)

    @pl.when(i == 0)
    def _init():
        acc_ref[...] = jnp.zeros_like(acc_ref)
        cnt_ref[...] = jnp.zeros_like(cnt_ref)

    w1 = w1_ref[...].astype(jnp.bfloat16)
    w2 = w2_ref[...].astype(jnp.bfloat16)
    b1 = b1_ref[...]
    b2 = b2_ref[...]
    ids = ids_ref[0, 0, :]
    seg = jax.lax.broadcasted_iota(jnp.int32, (N_BATCHES, BLK), 0)
    onehot = (seg == ids[None, :]).astype(jnp.float32)

    def half(lo):
        x = x_ref[pl.ds(lo, HALF), :].astype(jnp.bfloat16)
        h = jnp.dot(x, w1, preferred_element_type=jnp.float32) + b1
        h = jnp.maximum(h, 0.0).astype(jnp.bfloat16)
        y = jnp.dot(h, w2, preferred_element_type=jnp.float32) + b2
        return jnp.dot(
            onehot[:, lo : lo + HALF], y, preferred_element_type=jnp.float32
        )

    acc_ref[...] += half(0) + half(HALF)
    cnt_ref[...] += jnp.broadcast_to(
        jnp.sum(onehot, axis=1, keepdims=True), cnt_ref.shape
    )

    @pl.when(i == GRID - 1)
    def _fin():
        o_ref[...] = acc_ref[...] / jnp.maximum(cnt_ref[...], 1.0)


@jax.jit
def kernel(x_flat, batch, W1, b1, W2, b2):
    ids3 = batch.reshape(GRID, 1, BLK)
    b1r = b1.reshape(1, D_HIDDEN)
    b2r = b2.reshape(1, D_LATENT)
    return pl.pallas_call(
        _body,
        grid=(GRID,),
        in_specs=[
            pl.BlockSpec((BLK, D_IN), lambda i: (i, 0)),
            pl.BlockSpec((1, 1, BLK), lambda i: (i, 0, 0)),
            pl.BlockSpec((D_IN, D_HIDDEN), lambda i: (0, 0)),
            pl.BlockSpec((1, D_HIDDEN), lambda i: (0, 0)),
            pl.BlockSpec((D_HIDDEN, D_LATENT), lambda i: (0, 0)),
            pl.BlockSpec((1, D_LATENT), lambda i: (0, 0)),
        ],
        out_specs=pl.BlockSpec((N_BATCHES, D_LATENT), lambda i: (0, 0)),
        out_shape=jax.ShapeDtypeStruct((N_BATCHES, D_LATENT), jnp.float32),
        scratch_shapes=[
            pltpu.VMEM((N_BATCHES, D_LATENT), jnp.float32),
            pltpu.VMEM((N_BATCHES, D_LATENT), jnp.float32),
        ],
    )(x_flat, ids3, W1, b1r, W2, b2r)


# speedup vs baseline: 1.8563x; 1.0098x over previous
import functools

import jax
import jax.numpy as jnp
from jax.experimental import pallas as pl
from jax.experimental.pallas import tpu as pltpu

TOTAL_TOK = 16384
D_IN = 256
D_HIDDEN = 512
D_LATENT = 256
N_BATCHES = 16

BLK = 2048
HALF = 1024
GRID = TOTAL_TOK // BLK


def _body(x_ref, ids_ref, w1_ref, b1_ref, w2_ref, b2_ref, o_ref, acc_ref, cnt_ref):
    i = pl.program_id(0)

    @pl.when(i == 0)
    def _init():
        acc_ref[...] = jnp.zeros_like(acc_ref)
        cnt_ref[...] = jnp.zeros_like(cnt_ref)

    w1 = w1_ref[...].astype(jnp.bfloat16)
    w2 = w2_ref[...].astype(jnp.bfloat16)
    b1 = b1_ref[...]
    b2 = b2_ref[...]
    ids = ids_ref[0, 0, :]
    seg = jax.lax.broadcasted_iota(jnp.int32, (N_BATCHES, BLK), 0)
    onehot = (seg == ids[None, :]).astype(jnp.float32)

    def half(lo):
        x = x_ref[pl.ds(lo, HALF), :].astype(jnp.bfloat16)
        h = jnp.dot(x, w1, preferred_element_type=jnp.float32) + b1
        h = jnp.maximum(h, 0.0).astype(jnp.bfloat16)
        y = jnp.dot(h, w2, preferred_element_type=jnp.float32)
        return jnp.dot(
            onehot[:, lo : lo + HALF], y, preferred_element_type=jnp.float32
        )

    acc_ref[...] += sum(half(lo) for lo in range(0, BLK, HALF))
    cnt_ref[...] += jnp.broadcast_to(
        jnp.sum(onehot, axis=1, keepdims=True), cnt_ref.shape
    )

    @pl.when(i == GRID - 1)
    def _fin():
        cnt = cnt_ref[...]
        o_ref[...] = acc_ref[...] / jnp.maximum(cnt, 1.0) + jnp.where(
            cnt > 0.0, b2, 0.0
        )


@jax.jit
def kernel(x_flat, batch, W1, b1, W2, b2):
    ids3 = batch.reshape(GRID, 1, BLK)
    b1r = b1.reshape(1, D_HIDDEN)
    b2r = b2.reshape(1, D_LATENT)
    return pl.pallas_call(
        _body,
        grid=(GRID,),
        in_specs=[
            pl.BlockSpec((BLK, D_IN), lambda i: (i, 0)),
            pl.BlockSpec((1, 1, BLK), lambda i: (i, 0, 0)),
            pl.BlockSpec((D_IN, D_HIDDEN), lambda i: (0, 0)),
            pl.BlockSpec((1, D_HIDDEN), lambda i: (0, 0)),
            pl.BlockSpec((D_HIDDEN, D_LATENT), lambda i: (0, 0)),
            pl.BlockSpec((1, D_LATENT), lambda i: (0, 0)),
        ],
        out_specs=pl.BlockSpec((N_BATCHES, D_LATENT), lambda i: (0, 0)),
        out_shape=jax.ShapeDtypeStruct((N_BATCHES, D_LATENT), jnp.float32),
        scratch_shapes=[
            pltpu.VMEM((N_BATCHES, D_LATENT), jnp.float32),
            pltpu.VMEM((N_BATCHES, D_LATENT), jnp.float32),
        ],
    )(x_flat, ids3, W1, b1r, W2, b2r)


# speedup vs baseline: 1.9052x; 1.0264x over previous
import functools

import jax
import jax.numpy as jnp
from jax.experimental import pallas as pl
from jax.experimental.pallas import tpu as pltpu

TOTAL_TOK = 16384
D_IN = 256
D_HIDDEN = 512
D_LATENT = 256
N_BATCHES = 16

BLK = 2048
HALF = 1024
GRID = TOTAL_TOK // BLK


def _body(
    x_ref, ids_ref, w1_ref, b1_ref, w2_ref, b2_ref, o_ref, acc_ref, cnt_ref, y_ref
):
    i = pl.program_id(0)

    @pl.when(i == 0)
    def _init():
        acc_ref[...] = jnp.zeros_like(acc_ref)
        cnt_ref[...] = jnp.zeros_like(cnt_ref)

    b2 = b2_ref[...]

    @pl.when(i < GRID)
    def _mlp():
        w1 = w1_ref[...].astype(jnp.bfloat16)
        w2 = w2_ref[...].astype(jnp.bfloat16)
        b1 = b1_ref[...]
        for lo in (0, HALF):
            x = x_ref[pl.ds(lo, HALF), :].astype(jnp.bfloat16)
            h = jnp.dot(x, w1, preferred_element_type=jnp.float32) + b1
            h = jnp.maximum(h, 0.0).astype(jnp.bfloat16)
            y_ref[i % 2, pl.ds(lo, HALF), :] = jnp.dot(
                h, w2, preferred_element_type=jnp.float32
            )

    @pl.when(i > 0)
    def _pool():
        ids = ids_ref[0, 0, :]
        seg = jax.lax.broadcasted_iota(jnp.int32, (N_BATCHES, BLK), 0)
        onehot = (seg == ids[None, :]).astype(jnp.float32)
        y = y_ref[(i - 1) % 2]
        acc_ref[...] += jnp.dot(onehot, y, preferred_element_type=jnp.float32)
        cnt_ref[...] += jnp.broadcast_to(
            jnp.sum(onehot, axis=1, keepdims=True), cnt_ref.shape
        )

    @pl.when(i == GRID)
    def _fin():
        cnt = cnt_ref[...]
        o_ref[...] = acc_ref[...] / jnp.maximum(cnt, 1.0) + jnp.where(
            cnt > 0.0, b2, 0.0
        )


@jax.jit
def kernel(x_flat, batch, W1, b1, W2, b2):
    ids3 = batch.reshape(GRID, 1, BLK)
    b1r = b1.reshape(1, D_HIDDEN)
    b2r = b2.reshape(1, D_LATENT)
    last = GRID - 1
    return pl.pallas_call(
        _body,
        grid=(GRID + 1,),
        in_specs=[
            pl.BlockSpec((BLK, D_IN), lambda i: (jnp.minimum(i, last), 0)),
            pl.BlockSpec((1, 1, BLK), lambda i: (jnp.maximum(i - 1, 0), 0, 0)),
            pl.BlockSpec((D_IN, D_HIDDEN), lambda i: (0, 0)),
            pl.BlockSpec((1, D_HIDDEN), lambda i: (0, 0)),
            pl.BlockSpec((D_HIDDEN, D_LATENT), lambda i: (0, 0)),
            pl.BlockSpec((1, D_LATENT), lambda i: (0, 0)),
        ],
        out_specs=pl.BlockSpec((N_BATCHES, D_LATENT), lambda i: (0, 0)),
        out_shape=jax.ShapeDtypeStruct((N_BATCHES, D_LATENT), jnp.float32),
        scratch_shapes=[
            pltpu.VMEM((N_BATCHES, D_LATENT), jnp.float32),
            pltpu.VMEM((N_BATCHES, D_LATENT), jnp.float32),
            pltpu.VMEM((2, BLK, D_LATENT), jnp.float32),
        ],
    )(x_flat, ids3, W1, b1r, W2, b2r)


# speedup vs baseline: 2.0436x; 1.0726x over previous
import functools

import jax
import jax.numpy as jnp
from jax.experimental import pallas as pl
from jax.experimental.pallas import tpu as pltpu

TOTAL_TOK = 16384
D_IN = 256
D_HIDDEN = 512
D_LATENT = 256
N_BATCHES = 16

BLK = 4096
HALF = 1024
GRID = TOTAL_TOK // BLK


def _body(
    x_ref, ids_ref, w1_ref, b1_ref, w2_ref, b2_ref, o_ref, acc_ref, cnt_ref, y_ref
):
    i = pl.program_id(0)

    @pl.when(i == 0)
    def _init():
        acc_ref[...] = jnp.zeros_like(acc_ref)
        cnt_ref[...] = jnp.zeros_like(cnt_ref)

    b2 = b2_ref[...]

    @pl.when(i < GRID)
    def _mlp():
        w1 = w1_ref[...].astype(jnp.bfloat16)
        w2 = w2_ref[...].astype(jnp.bfloat16)
        b1 = b1_ref[...]
        for lo in range(0, BLK, HALF):
            x = x_ref[pl.ds(lo, HALF), :].astype(jnp.bfloat16)
            h = jnp.dot(x, w1, preferred_element_type=jnp.float32) + b1
            h = jnp.maximum(h, 0.0).astype(jnp.bfloat16)
            y_ref[i % 2, pl.ds(lo, HALF), :] = jnp.dot(
                h, w2, preferred_element_type=jnp.float32
            )

    @pl.when(i > 0)
    def _pool():
        ids = ids_ref[0, 0, :]
        seg = jax.lax.broadcasted_iota(jnp.int32, (N_BATCHES, BLK), 0)
        onehot = (seg == ids[None, :]).astype(jnp.float32)
        y = y_ref[(i - 1) % 2]
        acc_ref[...] += jnp.dot(onehot, y, preferred_element_type=jnp.float32)
        cnt_ref[...] += jnp.broadcast_to(
            jnp.sum(onehot, axis=1, keepdims=True), cnt_ref.shape
        )

    @pl.when(i == GRID)
    def _fin():
        cnt = cnt_ref[...]
        o_ref[...] = acc_ref[...] / jnp.maximum(cnt, 1.0) + jnp.where(
            cnt > 0.0, b2, 0.0
        )


@jax.jit
def kernel(x_flat, batch, W1, b1, W2, b2):
    ids3 = batch.reshape(GRID, 1, BLK)
    b1r = b1.reshape(1, D_HIDDEN)
    b2r = b2.reshape(1, D_LATENT)
    last = GRID - 1
    return pl.pallas_call(
        _body,
        grid=(GRID + 1,),
        in_specs=[
            pl.BlockSpec((BLK, D_IN), lambda i: (jnp.minimum(i, last), 0)),
            pl.BlockSpec((1, 1, BLK), lambda i: (jnp.maximum(i - 1, 0), 0, 0)),
            pl.BlockSpec((D_IN, D_HIDDEN), lambda i: (0, 0)),
            pl.BlockSpec((1, D_HIDDEN), lambda i: (0, 0)),
            pl.BlockSpec((D_HIDDEN, D_LATENT), lambda i: (0, 0)),
            pl.BlockSpec((1, D_LATENT), lambda i: (0, 0)),
        ],
        out_specs=pl.BlockSpec((N_BATCHES, D_LATENT), lambda i: (0, 0)),
        out_shape=jax.ShapeDtypeStruct((N_BATCHES, D_LATENT), jnp.float32),
        scratch_shapes=[
            pltpu.VMEM((N_BATCHES, D_LATENT), jnp.float32),
            pltpu.VMEM((N_BATCHES, D_LATENT), jnp.float32),
            pltpu.VMEM((2, BLK, D_LATENT), jnp.float32),
        ],
    )(x_flat, ids3, W1, b1r, W2, b2r)
